# baseline (speedup 1.0000x reference)
import functools

import jax
import jax.numpy as jnp
from jax import lax
from jax.experimental import pallas as pl
from jax.experimental.pallas import tpu as pltpu
from jax.experimental.pallas import tpu_sc as plsc

N_ENT = 1000000
N_REL = 1000
DIM = 64
LANES = 16

NUM_WORKERS = 32
CHUNK = 256


def _make_sc_kernel(n_total: int):
    per_w = n_total // NUM_WORKERS
    n_chunks = per_w // CHUNK
    mesh = plsc.VectorSubcoreMesh(core_axis_name="c", subcore_axis_name="s")

    @functools.partial(
        pl.kernel,
        out_type=jax.ShapeDtypeStruct((n_total,), jnp.float32),
        mesh=mesh,
        scratch_types=[
            pltpu.VMEM((N_REL, DIM), jnp.float32),
            pltpu.VMEM((CHUNK,), jnp.int32),
            pltpu.VMEM((CHUNK,), jnp.int32),
            pltpu.VMEM((CHUNK,), jnp.int32),
            pltpu.VMEM((CHUNK, DIM), jnp.float32),
            pltpu.VMEM((CHUNK, DIM), jnp.float32),
            pltpu.VMEM((CHUNK,), jnp.float32),
            pltpu.SemaphoreType.DMA,
        ],
        compiler_params=pltpu.CompilerParams(
            needs_layout_passes=False, use_tc_tiling_on_sc=False),
    )
    def sc_kernel(pred_hbm, subj_hbm, obj_hbm, ent_hbm, rel_hbm, out_hbm,
                  rel_v, sidx_v, oidx_v, pidx_v, he_v, te_v, outb_v, sem):
        wid = lax.axis_index("s") * 2 + lax.axis_index("c")
        pltpu.sync_copy(rel_hbm, rel_v)
        iota = lax.broadcasted_iota(jnp.int32, (LANES,), 0)

        def chunk_body(c, carry):
            base = wid * per_w + c * CHUNK
            pltpu.sync_copy(subj_hbm.at[pl.ds(base, CHUNK)], sidx_v)
            pltpu.sync_copy(obj_hbm.at[pl.ds(base, CHUNK)], oidx_v)
            pltpu.sync_copy(pred_hbm.at[pl.ds(base, CHUNK)], pidx_v)
            pltpu.async_copy(ent_hbm.at[sidx_v], he_v, sem).wait()
            pltpu.async_copy(ent_hbm.at[oidx_v], te_v, sem).wait()

            def group_body(g, carry2):
                rows = g * LANES + iota
                prow = pidx_v[pl.ds(g * LANES, LANES)]
                acc = jnp.zeros((LANES,), jnp.float32)
                for d in range(DIM):
                    dcol = jnp.full((LANES,), d, jnp.int32)
                    hv = plsc.load_gather(he_v, [rows, dcol])
                    tv = plsc.load_gather(te_v, [rows, dcol])
                    rv = plsc.load_gather(rel_v, [prow, dcol])
                    acc = acc + hv * tv * rv
                score = 1.0 / (1.0 + jnp.exp(-acc))
                outb_v[pl.ds(g * LANES, LANES)] = score
                return carry2

            lax.fori_loop(0, CHUNK // LANES, group_body, 0)
            pltpu.sync_copy(outb_v, out_hbm.at[pl.ds(base, CHUNK)])
            return carry

        lax.fori_loop(0, n_chunks, chunk_body, 0)

    return sc_kernel


def kernel(preds, subjs, objs, entity_emb, rel_emb):
    leading = preds.shape
    n_total = preds.size
    r = preds.reshape(-1).astype(jnp.int32)
    h = subjs.reshape(-1).astype(jnp.int32)
    t = objs.reshape(-1).astype(jnp.int32)
    sc = _make_sc_kernel(n_total)
    out = sc(r, h, t, entity_emb, rel_emb)
    return out.reshape(leading)


# speedup vs baseline: 1.0744x; 1.0744x over previous
import functools

import jax
import jax.numpy as jnp
from jax import lax
from jax.experimental import pallas as pl
from jax.experimental.pallas import tpu as pltpu
from jax.experimental.pallas import tpu_sc as plsc

N_ENT = 1000000
N_REL = 1000
DIM = 64
LANES = 16

NUM_WORKERS = 32
CHUNK = 128


def _make_sc_kernel(n_total: int):
    per_w = n_total // NUM_WORKERS
    n_chunks = per_w // CHUNK
    assert n_chunks % 2 == 0
    mesh = plsc.VectorSubcoreMesh(core_axis_name="c", subcore_axis_name="s")

    @functools.partial(
        pl.kernel,
        out_type=jax.ShapeDtypeStruct((n_total,), jnp.float32),
        mesh=mesh,
        scratch_types=[
            pltpu.VMEM((N_REL, DIM), jnp.float32),
            pltpu.VMEM((2, 3, CHUNK), jnp.int32),
            pltpu.VMEM((2, CHUNK, DIM), jnp.float32),
            pltpu.VMEM((2, CHUNK, DIM), jnp.float32),
            pltpu.VMEM((2, CHUNK), jnp.float32),
            pltpu.SemaphoreType.DMA,
            pltpu.SemaphoreType.DMA,
        ],
        compiler_params=pltpu.CompilerParams(
            needs_layout_passes=False,
            use_tc_tiling_on_sc=False,
            disable_bounds_checks=True,
        ),
    )
    def sc_kernel(idx_hbm, ent_hbm, rel_hbm, out_hbm,
                  rel_v, idx_v, he_v, te_v, outb_v, gsem0, gsem1):
        wid = lax.axis_index("s") * 2 + lax.axis_index("c")
        pltpu.sync_copy(rel_hbm, rel_v)
        iota = lax.broadcasted_iota(jnp.int32, (LANES,), 0)
        gsems = (gsem0, gsem1)

        def stage(chunk_id, b):
            ib, hb, tb, sem = idx_v.at[b], he_v.at[b], te_v.at[b], gsems[b]
            pltpu.sync_copy(idx_hbm.at[chunk_id], ib)
            pltpu.async_copy(ent_hbm.at[ib.at[0]], hb, sem)
            pltpu.async_copy(ent_hbm.at[ib.at[1]], tb, sem)

        def compute(chunk_id, b):
            ib, hb, tb, sem = idx_v.at[b], he_v.at[b], te_v.at[b], gsems[b]
            ob = outb_v.at[b]
            pltpu.make_async_copy(ent_hbm.at[ib.at[0]], hb, sem).wait()
            pltpu.make_async_copy(ent_hbm.at[ib.at[1]], tb, sem).wait()

            def group_body(g, carry):
                rows = g * LANES + iota
                prow = ib[2, pl.ds(g * LANES, LANES)]
                acc = jnp.zeros((LANES,), jnp.float32)
                for d in range(DIM):
                    dcol = jnp.full((LANES,), d, jnp.int32)
                    hv = plsc.load_gather(hb, [rows, dcol])
                    tv = plsc.load_gather(tb, [rows, dcol])
                    rv = plsc.load_gather(rel_v, [prow, dcol])
                    acc = acc + hv * tv * rv
                ob[pl.ds(g * LANES, LANES)] = 1.0 / (1.0 + jnp.exp(-acc))
                return carry

            lax.fori_loop(0, CHUNK // LANES, group_body, 0)
            base = (chunk_id - wid * n_chunks) * CHUNK + wid * per_w
            pltpu.sync_copy(ob, out_hbm.at[pl.ds(base, CHUNK)])

        first = wid * n_chunks
        stage(first, 0)

        def pair_body(i, carry):
            c0 = first + 2 * i
            stage(c0 + 1, 1)
            compute(c0, 0)
            stage(c0 + 2, 0)
            compute(c0 + 1, 1)
            return carry

        lax.fori_loop(0, n_chunks // 2, pair_body, 0)
        pltpu.make_async_copy(ent_hbm.at[idx_v.at[0].at[0]], he_v.at[0],
                              gsem0).wait()
        pltpu.make_async_copy(ent_hbm.at[idx_v.at[0].at[1]], te_v.at[0],
                              gsem0).wait()

    return sc_kernel


def kernel(preds, subjs, objs, entity_emb, rel_emb):
    leading = preds.shape
    n_total = preds.size
    r = preds.reshape(-1).astype(jnp.int32)
    h = subjs.reshape(-1).astype(jnp.int32)
    t = objs.reshape(-1).astype(jnp.int32)
    per_w = n_total // NUM_WORKERS
    n_chunks = per_w // CHUNK
    idx_all = (jnp.stack([h, t, r])
               .reshape(3, NUM_WORKERS * n_chunks, CHUNK)
               .transpose(1, 0, 2))
    idx_all = jnp.concatenate(
        [idx_all, jnp.zeros((1, 3, CHUNK), jnp.int32)], axis=0)
    sc = _make_sc_kernel(n_total)
    out = sc(idx_all, entity_emb, rel_emb)
    return out.reshape(leading)


# speedup vs baseline: 2.0014x; 1.8627x over previous
import functools

import jax
import jax.numpy as jnp
from jax import lax
from jax.experimental import pallas as pl
from jax.experimental.pallas import tpu as pltpu
from jax.experimental.pallas import tpu_sc as plsc

N_ENT = 1000000
N_REL = 1000
DIM = 64
LANES = 16

NUM_WORKERS = 32
CHUNK = 128


def _make_sc_kernel(n_total: int):
    per_w = n_total // NUM_WORKERS
    n_chunks = per_w // CHUNK
    assert n_chunks % 2 == 0
    mesh = plsc.VectorSubcoreMesh(core_axis_name="c", subcore_axis_name="s")

    @functools.partial(
        pl.kernel,
        out_type=jax.ShapeDtypeStruct((n_total,), jnp.float32),
        mesh=mesh,
        scratch_types=[
            pltpu.VMEM((N_REL, DIM), jnp.float32),
            pltpu.VMEM((2, 3, CHUNK), jnp.int32),
            pltpu.VMEM((2, CHUNK, DIM), jnp.float32),
            pltpu.VMEM((2, CHUNK, DIM), jnp.float32),
            pltpu.VMEM((2, CHUNK), jnp.float32),
            pltpu.SemaphoreType.DMA,
            pltpu.SemaphoreType.DMA,
        ],
        compiler_params=pltpu.CompilerParams(
            needs_layout_passes=False,
            use_tc_tiling_on_sc=False,
            disable_bounds_checks=True,
        ),
    )
    def sc_kernel(idx_hbm, ent_hbm, rel_hbm, out_hbm,
                  rel_v, idx_v, he_v, te_v, outb_v, gsem0, gsem1):
        wid = lax.axis_index("s") * 2 + lax.axis_index("c")
        pltpu.sync_copy(rel_hbm, rel_v)
        iota = lax.broadcasted_iota(jnp.int32, (LANES,), 0)
        gsems = (gsem0, gsem1)

        def stage(chunk_id, b):
            ib, hb, tb, sem = idx_v.at[b], he_v.at[b], te_v.at[b], gsems[b]
            pltpu.sync_copy(idx_hbm.at[chunk_id], ib)
            pltpu.async_copy(ent_hbm.at[ib.at[0]], hb, sem)
            pltpu.async_copy(ent_hbm.at[ib.at[1]], tb, sem)

        def compute(chunk_id, b):
            ib, hb, tb, sem = idx_v.at[b], he_v.at[b], te_v.at[b], gsems[b]
            ob = outb_v.at[b]
            pltpu.make_async_copy(ent_hbm.at[ib.at[0]], hb, sem).wait()
            pltpu.make_async_copy(ent_hbm.at[ib.at[1]], tb, sem).wait()

            def group_body(g, carry):
                rows = g * LANES + iota
                prow = ib[2, pl.ds(g * LANES, LANES)]
                accs = [jnp.zeros((LANES,), jnp.float32) for _ in range(4)]
                for d in range(DIM):
                    dcol = jnp.bitwise_and(iota + d, DIM - 1)
                    hv = plsc.load_gather(hb, [rows, dcol])
                    tv = plsc.load_gather(tb, [rows, dcol])
                    rv = plsc.load_gather(rel_v, [prow, dcol])
                    accs[d % 4] = accs[d % 4] + hv * tv * rv
                acc = (accs[0] + accs[1]) + (accs[2] + accs[3])
                ob[pl.ds(g * LANES, LANES)] = 1.0 / (1.0 + jnp.exp(-acc))
                return carry

            lax.fori_loop(0, CHUNK // LANES, group_body, 0)
            base = (chunk_id - wid * n_chunks) * CHUNK + wid * per_w
            pltpu.sync_copy(ob, out_hbm.at[pl.ds(base, CHUNK)])

        first = wid * n_chunks
        stage(first, 0)

        def pair_body(i, carry):
            c0 = first + 2 * i
            stage(c0 + 1, 1)
            compute(c0, 0)
            stage(c0 + 2, 0)
            compute(c0 + 1, 1)
            return carry

        lax.fori_loop(0, n_chunks // 2, pair_body, 0)
        pltpu.make_async_copy(ent_hbm.at[idx_v.at[0].at[0]], he_v.at[0],
                              gsem0).wait()
        pltpu.make_async_copy(ent_hbm.at[idx_v.at[0].at[1]], te_v.at[0],
                              gsem0).wait()

    return sc_kernel


def kernel(preds, subjs, objs, entity_emb, rel_emb):
    leading = preds.shape
    n_total = preds.size
    r = preds.reshape(-1).astype(jnp.int32)
    h = subjs.reshape(-1).astype(jnp.int32)
    t = objs.reshape(-1).astype(jnp.int32)
    per_w = n_total // NUM_WORKERS
    n_chunks = per_w // CHUNK
    idx_all = (jnp.stack([h, t, r])
               .reshape(3, NUM_WORKERS * n_chunks, CHUNK)
               .transpose(1, 0, 2))
    idx_all = jnp.concatenate(
        [idx_all, jnp.zeros((1, 3, CHUNK), jnp.int32)], axis=0)
    sc = _make_sc_kernel(n_total)
    out = sc(idx_all, entity_emb, rel_emb)
    return out.reshape(leading)
